# baseline (speedup 1.0000x reference)
import jax
import jax.numpy as jnp
from jax import lax
from jax.experimental import pallas as pl
from jax.experimental.pallas import tpu as pltpu
from jax.experimental.pallas import tpu_sc as plsc

_NBUF = 3
_CHUNK = 32


def _sc_body(in_hbm, out_hbm, *scratch):
    bufs = scratch[:_NBUF]
    rsems = scratch[_NBUF:2 * _NBUF]
    wsems = scratch[2 * _NBUF:3 * _NBUF]

    info = plsc.get_sparse_core_info()
    n_workers = info.num_cores * info.num_subcores
    rows = in_hbm.shape[0]
    per_worker = rows // n_workers
    nchunks = per_worker // _CHUNK

    wid = lax.axis_index("s") * info.num_cores + lax.axis_index("c")
    base = wid * per_worker

    def read(i):
        b = i % _NBUF
        return pltpu.make_async_copy(
            in_hbm.at[pl.ds(base + i * _CHUNK, _CHUNK), :], bufs[b], rsems[b])

    def write(i):
        b = i % _NBUF
        return pltpu.make_async_copy(
            bufs[b], out_hbm.at[pl.ds(base + i * _CHUNK, _CHUNK), :], wsems[b])

    for i in range(min(_NBUF, nchunks)):
        read(i).start()
    for i in range(nchunks):
        read(i).wait()
        write(i).start()
        j = i + _NBUF
        if j < nchunks:
            write(i).wait()
            read(j).start()
    for i in range(max(0, nchunks - _NBUF), nchunks):
        write(i).wait()


def kernel(seq_len, matrix):
    del seq_len
    rows, cols = matrix.shape
    mesh = plsc.VectorSubcoreMesh(core_axis_name="c", subcore_axis_name="s")
    sc_copy = pl.kernel(
        _sc_body,
        out_type=jax.ShapeDtypeStruct((rows, cols), matrix.dtype),
        mesh=mesh,
        scratch_types=(
            [pltpu.VMEM((_CHUNK, cols), matrix.dtype)] * _NBUF
            + [pltpu.SemaphoreType.DMA] * (2 * _NBUF)
        ),
    )
    return sc_copy(matrix)
